# baseline (speedup 1.0000x reference)
import functools

import jax
import jax.numpy as jnp
from jax import lax
from jax.experimental import pallas as pl
from jax.experimental.pallas import tpu as pltpu
from jax.experimental.pallas import tpu_sc as plsc


def _dot(a, b, dims):
    return jax.lax.dot_general(a, b, (dims, ((), ())),
                               preferred_element_type=jnp.float32)


def _vq_block_kernel(x_ref, cb_ref, idx_ref, counts_ref, diff_ref,
                     ent_ref, w2_ref, e2_ref, msum_ref,
                     *, blk, n, c, bt, nblocks):
    step = pl.program_id(0)

    @pl.when(step == 0)
    def _init():
        w = cb_ref[...]
        w2_ref[...] = -2.0 * w
        e2_ref[...] = jnp.sum(w * w, axis=1, keepdims=True).reshape(1, n)
        counts_ref[...] = jnp.zeros_like(counts_ref)
        msum_ref[0] = 0.0

    x = x_ref[...]
    xwt = _dot(x, w2_ref[...], ((1,), (1,)))
    e2 = e2_ref[...]
    lanes = 128
    nch = n // lanes
    run_val = xwt[:, :lanes] + e2[:, :lanes]
    run_src = jnp.zeros((blk, lanes), dtype=jnp.int32)
    for k in range(1, nch):
        v = xwt[:, k * lanes:(k + 1) * lanes] + e2[:, k * lanes:(k + 1) * lanes]
        better = v < run_val
        run_val = jnp.where(better, v, run_val)
        run_src = jnp.where(better, k, run_src)
    min_d = jnp.min(run_val, axis=1, keepdims=True)
    lane = jax.lax.broadcasted_iota(jnp.int32, (blk, lanes), 1)
    j_lane = run_src * lanes + lane
    idx = jnp.min(jnp.where(run_val == min_d, j_lane, n), axis=1,
                  keepdims=True)
    iota = jax.lax.broadcasted_iota(jnp.int32, (blk, n), 1)
    one_hot = (iota == idx).astype(jnp.float32)

    idx_ref[...] = idx
    ones_row = jnp.ones((1, blk), dtype=jnp.float32)
    counts_ref[...] += _dot(ones_row, one_hot, ((1,), (0,)))
    ones_col = jnp.ones((c, 1), dtype=jnp.float32)
    x2row = _dot(x * x, ones_col, ((1,), (0,)))
    msum_ref[0] += _dot(jnp.ones((1, blk), jnp.float32), min_d + x2row,
                        ((1,), (0,)))[0, 0]

    @pl.when(step == nblocks - 1)
    def _finish():
        counts = counts_ref[...]
        p = counts / bt
        ent = jnp.exp(-jnp.sum(p * jnp.log(p + 1e-10)))
        ent_ref[...] = jnp.full((1, 1), ent, dtype=jnp.float32)
        diff_ref[...] = jnp.full((1, 1), msum_ref[0] / (bt * c),
                                 dtype=jnp.float32)


def _sc_gather(bt, c, chunk=256):
    info = plsc.get_sparse_core_info()
    nw = info.num_cores * info.num_subcores
    b_per_w = bt // nw
    mesh = plsc.VectorSubcoreMesh(core_axis_name="c", subcore_axis_name="s")

    @functools.partial(
        pl.kernel, mesh=mesh,
        out_type=jax.ShapeDtypeStruct((bt, c), jnp.float32),
        scratch_types=[
            pltpu.VMEM((chunk,), jnp.int32),
            pltpu.VMEM((chunk, c), jnp.float32),
            pltpu.SemaphoreType.DMA,
        ],
    )
    def gather_kernel(cb_hbm, idx_hbm, out_hbm, idx_v, rows_v, sem):
        wid = lax.axis_index("s") * info.num_cores + lax.axis_index("c")
        base = wid * b_per_w
        for j in range(b_per_w // chunk):
            off = base + j * chunk
            pltpu.sync_copy(idx_hbm.at[pl.ds(off, chunk)], idx_v)
            pltpu.async_copy(cb_hbm.at[idx_v], rows_v, sem).wait()
            pltpu.sync_copy(rows_v, out_hbm.at[pl.ds(off, chunk)])

    return gather_kernel


def kernel(x, codebook, steps):
    B, T, C = x.shape
    N = codebook.shape[0]
    BT = B * T
    BLK = 2048
    nblocks = BT // BLK
    x_flat = x.reshape(BT, C)

    kfn = functools.partial(_vq_block_kernel, blk=BLK, n=N, c=C, bt=BT,
                            nblocks=nblocks)
    idx, counts, diff, ent = pl.pallas_call(
        kfn,
        grid=(nblocks,),
        in_specs=[
            pl.BlockSpec((BLK, C), lambda i: (i, 0)),
            pl.BlockSpec((N, C), lambda i: (0, 0)),
        ],
        out_specs=[
            pl.BlockSpec((BLK, 1), lambda i: (i, 0)),
            pl.BlockSpec((1, N), lambda i: (0, 0)),
            pl.BlockSpec((1, 1), lambda i: (0, 0)),
            pl.BlockSpec((1, 1), lambda i: (0, 0)),
        ],
        out_shape=[
            jax.ShapeDtypeStruct((BT, 1), jnp.int32),
            jax.ShapeDtypeStruct((1, N), jnp.float32),
            jax.ShapeDtypeStruct((1, 1), jnp.float32),
            jax.ShapeDtypeStruct((1, 1), jnp.float32),
        ],
        scratch_shapes=[
            pltpu.VMEM((N, C), jnp.float32),
            pltpu.VMEM((1, N), jnp.float32),
            pltpu.SMEM((1,), jnp.float32),
        ],
    )(x_flat, codebook)

    q = _sc_gather(BT, C)(codebook, idx[:, 0])
    return idx, q.reshape(B, T, C), diff[0, 0], ent[0, 0]


# speedup vs baseline: 1.6557x; 1.6557x over previous
import functools

import jax
import jax.numpy as jnp
from jax.experimental import pallas as pl
from jax.experimental.pallas import tpu as pltpu


def _dot(a, b, dims):
    return jax.lax.dot_general(a, b, (dims, ((), ())),
                               preferred_element_type=jnp.float32)


def _vq_block_kernel(x_ref, cb_ref, idx_ref, q_ref, counts_ref, diff_ref,
                     ent_ref, w2_ref, e2_ref, msum_ref,
                     *, blk, n, c, bt, nblocks):
    step = pl.program_id(0)

    @pl.when(step == 0)
    def _init():
        w = cb_ref[...]
        w2_ref[...] = -2.0 * w
        e2_ref[...] = jnp.sum(w * w, axis=1, keepdims=True).reshape(1, n)
        counts_ref[...] = jnp.zeros_like(counts_ref)
        msum_ref[0] = 0.0

    x = x_ref[...]
    xwt = _dot(x, w2_ref[...], ((1,), (1,)))
    e2 = e2_ref[...]
    lanes = 128
    nch = n // lanes
    run_val = xwt[:, :lanes] + e2[:, :lanes]
    run_src = jnp.zeros((blk, lanes), dtype=jnp.int32)
    for k in range(1, nch):
        v = xwt[:, k * lanes:(k + 1) * lanes] + e2[:, k * lanes:(k + 1) * lanes]
        better = v < run_val
        run_val = jnp.where(better, v, run_val)
        run_src = jnp.where(better, k, run_src)
    min_d = jnp.min(run_val, axis=1, keepdims=True)
    lane = jax.lax.broadcasted_iota(jnp.int32, (blk, lanes), 1)
    j_lane = run_src * lanes + lane
    idx = jnp.min(jnp.where(run_val == min_d, j_lane, n), axis=1,
                  keepdims=True)
    iota = jax.lax.broadcasted_iota(jnp.int32, (blk, n), 1)
    one_hot = (iota == idx).astype(jnp.float32)
    q = _dot(one_hot, cb_ref[...], ((1,), (0,)))

    idx_ref[...] = idx
    q_ref[...] = q
    ones_row = jnp.ones((1, blk), dtype=jnp.float32)
    counts_ref[...] += _dot(ones_row, one_hot, ((1,), (0,)))
    ones_col = jnp.ones((c, 1), dtype=jnp.float32)
    x2row = _dot(x * x, ones_col, ((1,), (0,)))
    msum_ref[0] += _dot(jnp.ones((1, blk), jnp.float32), min_d + x2row,
                        ((1,), (0,)))[0, 0]

    @pl.when(step == nblocks - 1)
    def _finish():
        counts = counts_ref[...]
        p = counts / bt
        ent = jnp.exp(-jnp.sum(p * jnp.log(p + 1e-10)))
        ent_ref[...] = jnp.full((1, 1), ent, dtype=jnp.float32)
        diff_ref[...] = jnp.full((1, 1), msum_ref[0] / (bt * c),
                                 dtype=jnp.float32)


def kernel(x, codebook, steps):
    B, T, C = x.shape
    N = codebook.shape[0]
    BT = B * T
    BLK = 2048
    nblocks = BT // BLK
    x_flat = x.reshape(BT, C)

    kfn = functools.partial(_vq_block_kernel, blk=BLK, n=N, c=C, bt=BT,
                            nblocks=nblocks)
    idx, q, counts, diff, ent = pl.pallas_call(
        kfn,
        grid=(nblocks,),
        in_specs=[
            pl.BlockSpec((BLK, C), lambda i: (i, 0)),
            pl.BlockSpec((N, C), lambda i: (0, 0)),
        ],
        out_specs=[
            pl.BlockSpec((BLK, 1), lambda i: (i, 0)),
            pl.BlockSpec((BLK, C), lambda i: (i, 0)),
            pl.BlockSpec((1, N), lambda i: (0, 0)),
            pl.BlockSpec((1, 1), lambda i: (0, 0)),
            pl.BlockSpec((1, 1), lambda i: (0, 0)),
        ],
        out_shape=[
            jax.ShapeDtypeStruct((BT, 1), jnp.int32),
            jax.ShapeDtypeStruct((BT, C), jnp.float32),
            jax.ShapeDtypeStruct((1, N), jnp.float32),
            jax.ShapeDtypeStruct((1, 1), jnp.float32),
            jax.ShapeDtypeStruct((1, 1), jnp.float32),
        ],
        scratch_shapes=[
            pltpu.VMEM((N, C), jnp.float32),
            pltpu.VMEM((1, N), jnp.float32),
            pltpu.SMEM((1,), jnp.float32),
        ],
    )(x_flat, codebook)

    return idx, q.reshape(B, T, C), diff[0, 0], ent[0, 0]
